# baseline (speedup 1.0000x reference)
import functools

import jax
import jax.numpy as jnp
from jax import lax
from jax.experimental import pallas as pl
from jax.experimental.pallas import tpu as pltpu
from jax.experimental.pallas import tpu_sc as plsc

NUM_CLASSES = 10
K = 2000
N = 16
NC, NS, L = 2, 16, 16
NW = NC * NS
BS = 64
JB = 32
M = (NUM_CLASSES - 1) * N
CH = 3 * 32
WDIM = 32
WQ = 8
WSPLIT = WDIM // WQ
UNITS = CH * WSPLIT * (BS // JB)
UNITS_PER_W = UNITS // NW
JC = BS * NUM_CLASSES


def _gather_body(bank_hbm, labelrep_hbm, sel_hbm, out_hbm,
                 labelrep_v, sel_v, midx_v, slabA_v, slabB_v,
                 stage_v, semA, semB, osem):
    w = lax.axis_index("s") * NC + lax.axis_index("c")
    pltpu.sync_copy(labelrep_hbm, labelrep_v)
    pltpu.sync_copy(sel_hbm, sel_v)
    lanes = lax.iota(jnp.int32, L)

    @pl.loop(0, JC)
    def _tab(c2):
        j = c2 // NUM_CLASSES
        cls = c2 % NUM_CLASSES
        lbl = labelrep_v[pl.ds(j * L, L)]
        clsv = jnp.full((L,), cls, jnp.int32)
        gtv = (clsv > lbl).astype(jnp.int32)
        eqv = (clsv == lbl).astype(jnp.int32)
        midx_v[pl.ds(c2 * L, L)] = (clsv - gtv) * N + lanes + eqv * 4 * M

    slabs = (slabA_v, slabB_v)
    sems = (semA, semB)

    def slab_start(cls, ch, w0, b):
        pltpu.async_copy(bank_hbm.at[cls * CH + ch, pl.ds(w0, WQ), :],
                         slabs[b], sems[b])

    def slab_wait(ch, w0, b):
        pltpu.make_async_copy(bank_hbm.at[ch, pl.ds(w0, WQ), :],
                              slabs[b], sems[b]).wait()

    wfull = [jnp.full((L,), wr, jnp.int32) for wr in range(WQ)]

    @pl.loop(0, UNITS_PER_W)
    def _pos(i):
        u = w * UNITS_PER_W + i
        ch = u // (WSPLIT * 2)
        rest = u % (WSPLIT * 2)
        j0 = (rest % 2) * JB
        w0 = (rest // 2) * WQ

        slab_start(0, ch, w0, 0)

        @pl.when(i > 0)
        def _drain_prev():
            @pl.loop(0, JB)
            def _d(jj):
                pltpu.make_async_copy(
                    stage_v.at[0], out_hbm.at[0, 0, pl.ds(0, WQ), :],
                    osem).wait()

        for cls in range(NUM_CLASSES):
            b = cls % 2
            slab_wait(ch, w0, b)
            if cls + 1 < NUM_CLASSES:
                slab_start(cls + 1, ch, w0, 1 - b)

            last = cls == NUM_CLASSES - 1

            @plsc.parallel_loop(0, JB, unroll=4)
            def _sample(jj):
                j = j0 + jj
                base = (j * NUM_CLASSES + cls) * L
                kvec = sel_v[pl.ds(base, L)]
                midx = midx_v[pl.ds(base, L)]
                mk = midx < M
                jful = jnp.full((L,), jj, jnp.int32)
                for wr in range(WQ):
                    vals = plsc.load_gather(slabs[b], [wfull[wr], kvec])
                    plsc.store_scatter(stage_v, [jful, wfull[wr], midx],
                                       vals, mask=mk)
                if last:
                    pltpu.async_copy(
                        stage_v.at[jj],
                        out_hbm.at[j0 + jj, ch, pl.ds(w0, WQ), :], osem)

    _final_drain(stage_v, out_hbm, osem)


def _final_drain(stage_v, out_hbm, osem):
    @pl.loop(0, JB)
    def _d(jj):
        pltpu.make_async_copy(stage_v.at[0],
                              out_hbm.at[0, 0, pl.ds(0, WQ), :], osem).wait()


def kernel(x_s, label, confounder_queue, sel_idx):
    bs = x_s.shape[0]
    C, H, W = (confounder_queue.shape[2], confounder_queue.shape[3],
               confounder_queue.shape[4])
    bank = confounder_queue.transpose(0, 2, 3, 4, 1).reshape(
        NUM_CLASSES * C * H, W, K)
    sel_flat = sel_idx.reshape(-1).astype(jnp.int32)
    label_rep = jnp.repeat(label.astype(jnp.int32), L)
    mesh = plsc.VectorSubcoreMesh(
        core_axis_name="c", subcore_axis_name="s",
        num_cores=NC, num_subcores=NS)
    run = pl.kernel(
        _gather_body,
        out_type=jax.ShapeDtypeStruct((bs, C * H, W, M), jnp.float32),
        mesh=mesh,
        compiler_params=pltpu.CompilerParams(
            needs_layout_passes=False, use_tc_tiling_on_sc=True,
            disable_bounds_checks=True),
        scratch_types=[
            pltpu.VMEM((BS * L,), jnp.int32),
            pltpu.VMEM((JC * L,), jnp.int32),
            pltpu.VMEM((JC * L,), jnp.int32),
            pltpu.VMEM((WQ, K), jnp.float32),
            pltpu.VMEM((WQ, K), jnp.float32),
            pltpu.VMEM((JB, WQ, M), jnp.float32),
            pltpu.SemaphoreType.DMA,
            pltpu.SemaphoreType.DMA,
            pltpu.SemaphoreType.DMA,
        ],
    )
    out = run(bank, label_rep, sel_flat)
    return out.reshape(bs, C, H, W, M).transpose(0, 4, 1, 2, 3)


# speedup vs baseline: 1.0063x; 1.0063x over previous
import functools

import jax
import jax.numpy as jnp
from jax import lax
from jax.experimental import pallas as pl
from jax.experimental.pallas import tpu as pltpu
from jax.experimental.pallas import tpu_sc as plsc

NUM_CLASSES = 10
K = 2000
N = 16
NC, NS, L = 2, 16, 16
NW = NC * NS
BS = 64
JB = 32
M = (NUM_CLASSES - 1) * N
CH = 3 * 32
WDIM = 32
WQ = 8
WSPLIT = WDIM // WQ
UNITS = CH * WSPLIT * (BS // JB)
UNITS_PER_W = UNITS // NW
JC = BS * NUM_CLASSES


def _gather_body(bank_hbm, labelrep_hbm, sel_hbm, out_hbm,
                 labelrep_v, sel_v, midx_v, slabA_v, slabB_v,
                 stage_v, semA, semB, osem):
    w = lax.axis_index("s") * NC + lax.axis_index("c")
    pltpu.sync_copy(labelrep_hbm, labelrep_v)
    pltpu.sync_copy(sel_hbm, sel_v)
    lanes = lax.iota(jnp.int32, L)

    @pl.loop(0, JC)
    def _tab(c2):
        j = c2 // NUM_CLASSES
        cls = c2 % NUM_CLASSES
        lbl = labelrep_v[pl.ds(j * L, L)]
        clsv = jnp.full((L,), cls, jnp.int32)
        gtv = (clsv > lbl).astype(jnp.int32)
        eqv = (clsv == lbl).astype(jnp.int32)
        midx_v[pl.ds(c2 * L, L)] = (clsv - gtv) * N + lanes + eqv * 4 * M

    slabs = (slabA_v, slabB_v)
    sems = (semA, semB)

    def slab_start(cls, ch, w0, b):
        pltpu.async_copy(bank_hbm.at[cls * CH + ch, pl.ds(w0, WQ), :],
                         slabs[b], sems[b])

    def slab_wait(ch, w0, b):
        pltpu.make_async_copy(bank_hbm.at[ch, pl.ds(w0, WQ), :],
                              slabs[b], sems[b]).wait()

    wfull = [jnp.full((L,), wr, jnp.int32) for wr in range(WQ)]

    @pl.loop(0, UNITS_PER_W)
    def _pos(i):
        u = w * UNITS_PER_W + i
        ch = u // (WSPLIT * 2)
        rest = u % (WSPLIT * 2)
        j0 = (rest % 2) * JB
        w0 = (rest // 2) * WQ

        slab_start(0, ch, w0, 0)

        @pl.when(i > 0)
        def _drain_prev():
            @pl.loop(0, JB)
            def _d(jj):
                pltpu.make_async_copy(
                    stage_v.at[0], out_hbm.at[0, 0, pl.ds(0, WQ), :],
                    osem).wait()

        for cls in range(NUM_CLASSES):
            b = cls % 2
            slab_wait(ch, w0, b)
            if cls + 1 < NUM_CLASSES:
                slab_start(cls + 1, ch, w0, 1 - b)

            last = cls == NUM_CLASSES - 1

            @plsc.parallel_loop(0, JB, unroll=2)
            def _sample(jj):
                j = j0 + jj
                base = (j * NUM_CLASSES + cls) * L
                kvec = sel_v[pl.ds(base, L)]
                midx = midx_v[pl.ds(base, L)]
                mk = midx < M
                jful = jnp.full((L,), jj, jnp.int32)
                for wr in range(WQ):
                    vals = plsc.load_gather(slabs[b], [wfull[wr], kvec])
                    plsc.store_scatter(stage_v, [jful, wfull[wr], midx],
                                       vals, mask=mk)
                if last:
                    pltpu.async_copy(
                        stage_v.at[jj],
                        out_hbm.at[j0 + jj, ch, pl.ds(w0, WQ), :], osem)

    _final_drain(stage_v, out_hbm, osem)


def _final_drain(stage_v, out_hbm, osem):
    @pl.loop(0, JB)
    def _d(jj):
        pltpu.make_async_copy(stage_v.at[0],
                              out_hbm.at[0, 0, pl.ds(0, WQ), :], osem).wait()


def kernel(x_s, label, confounder_queue, sel_idx):
    bs = x_s.shape[0]
    C, H, W = (confounder_queue.shape[2], confounder_queue.shape[3],
               confounder_queue.shape[4])
    bank = confounder_queue.transpose(0, 2, 3, 4, 1).reshape(
        NUM_CLASSES * C * H, W, K)
    sel_flat = sel_idx.reshape(-1).astype(jnp.int32)
    label_rep = jnp.repeat(label.astype(jnp.int32), L)
    mesh = plsc.VectorSubcoreMesh(
        core_axis_name="c", subcore_axis_name="s",
        num_cores=NC, num_subcores=NS)
    run = pl.kernel(
        _gather_body,
        out_type=jax.ShapeDtypeStruct((bs, C * H, W, M), jnp.float32),
        mesh=mesh,
        compiler_params=pltpu.CompilerParams(
            needs_layout_passes=False, use_tc_tiling_on_sc=True,
            disable_bounds_checks=True),
        scratch_types=[
            pltpu.VMEM((BS * L,), jnp.int32),
            pltpu.VMEM((JC * L,), jnp.int32),
            pltpu.VMEM((JC * L,), jnp.int32),
            pltpu.VMEM((WQ, K), jnp.float32),
            pltpu.VMEM((WQ, K), jnp.float32),
            pltpu.VMEM((JB, WQ, M), jnp.float32),
            pltpu.SemaphoreType.DMA,
            pltpu.SemaphoreType.DMA,
            pltpu.SemaphoreType.DMA,
        ],
    )
    out = run(bank, label_rep, sel_flat)
    return out.reshape(bs, C, H, W, M).transpose(0, 4, 1, 2, 3)


# speedup vs baseline: 1.0068x; 1.0005x over previous
import functools

import jax
import jax.numpy as jnp
from jax import lax
from jax.experimental import pallas as pl
from jax.experimental.pallas import tpu as pltpu
from jax.experimental.pallas import tpu_sc as plsc

NUM_CLASSES = 10
K = 2000
N = 16
NC, NS, L = 2, 16, 16
NW = NC * NS
BS = 64
JB = 32
M = (NUM_CLASSES - 1) * N
CH = 3 * 32
WDIM = 32
WQ = 8
WSPLIT = WDIM // WQ
UNITS = CH * WSPLIT * (BS // JB)
UNITS_PER_W = UNITS // NW
JC = BS * NUM_CLASSES


def _gather_body(bank_hbm, labelrep_hbm, sel_hbm, out_hbm,
                 labelrep_v, sel_v, midx_v, slabA_v, slabB_v,
                 stage_v, semA, semB, osem):
    w = lax.axis_index("s") * NC + lax.axis_index("c")
    pltpu.sync_copy(labelrep_hbm, labelrep_v)
    pltpu.sync_copy(sel_hbm, sel_v)
    lanes = lax.iota(jnp.int32, L)

    @pl.loop(0, JC)
    def _tab(c2):
        j = c2 // NUM_CLASSES
        cls = c2 % NUM_CLASSES
        lbl = labelrep_v[pl.ds(j * L, L)]
        clsv = jnp.full((L,), cls, jnp.int32)
        gtv = (clsv > lbl).astype(jnp.int32)
        eqv = (clsv == lbl).astype(jnp.int32)
        midx_v[pl.ds(c2 * L, L)] = ((clsv - gtv) * N + lanes
                                    + eqv * (M - (clsv - gtv) * N))

    slabs = (slabA_v, slabB_v)
    sems = (semA, semB)

    def slab_start(cls, ch, w0, b):
        pltpu.async_copy(bank_hbm.at[cls * CH + ch, pl.ds(w0, WQ), :],
                         slabs[b], sems[b])

    def slab_wait(ch, w0, b):
        pltpu.make_async_copy(bank_hbm.at[ch, pl.ds(w0, WQ), :],
                              slabs[b], sems[b]).wait()

    wfull = [jnp.full((L,), wr, jnp.int32) for wr in range(WQ)]

    @pl.loop(0, UNITS_PER_W)
    def _pos(i):
        u = w * UNITS_PER_W + i
        ch = u // (WSPLIT * 2)
        rest = u % (WSPLIT * 2)
        j0 = (rest % 2) * JB
        w0 = (rest // 2) * WQ

        slab_start(0, ch, w0, 0)

        @pl.when(i > 0)
        def _drain_prev():
            @pl.loop(0, JB)
            def _d(jj):
                pltpu.make_async_copy(
                    stage_v.at[0], out_hbm.at[0, 0, pl.ds(0, WQ), :],
                    osem).wait()

        for cls in range(NUM_CLASSES):
            b = cls % 2
            slab_wait(ch, w0, b)
            if cls + 1 < NUM_CLASSES:
                slab_start(cls + 1, ch, w0, 1 - b)

            last = cls == NUM_CLASSES - 1

            @plsc.parallel_loop(0, JB, unroll=2)
            def _sample(jj):
                j = j0 + jj
                base = (j * NUM_CLASSES + cls) * L
                kvec = sel_v[pl.ds(base, L)]
                midx = midx_v[pl.ds(base, L)]
                jful = jnp.full((L,), jj, jnp.int32)
                for wr in range(WQ):
                    vals = plsc.load_gather(slabs[b], [wfull[wr], kvec])
                    plsc.store_scatter(stage_v, [jful, wfull[wr], midx],
                                       vals)
                if last:
                    pltpu.async_copy(
                        stage_v.at[jj],
                        out_hbm.at[j0 + jj, ch, pl.ds(w0, WQ), :], osem)

    _final_drain(stage_v, out_hbm, osem)


def _final_drain(stage_v, out_hbm, osem):
    @pl.loop(0, JB)
    def _d(jj):
        pltpu.make_async_copy(stage_v.at[0],
                              out_hbm.at[0, 0, pl.ds(0, WQ), :], osem).wait()


def kernel(x_s, label, confounder_queue, sel_idx):
    bs = x_s.shape[0]
    C, H, W = (confounder_queue.shape[2], confounder_queue.shape[3],
               confounder_queue.shape[4])
    bank = confounder_queue.transpose(0, 2, 3, 4, 1).reshape(
        NUM_CLASSES * C * H, W, K)
    sel_flat = sel_idx.reshape(-1).astype(jnp.int32)
    label_rep = jnp.repeat(label.astype(jnp.int32), L)
    mesh = plsc.VectorSubcoreMesh(
        core_axis_name="c", subcore_axis_name="s",
        num_cores=NC, num_subcores=NS)
    run = pl.kernel(
        _gather_body,
        out_type=jax.ShapeDtypeStruct((bs, C * H, W, M), jnp.float32),
        mesh=mesh,
        compiler_params=pltpu.CompilerParams(
            needs_layout_passes=False, use_tc_tiling_on_sc=True,
            disable_bounds_checks=True),
        scratch_types=[
            pltpu.VMEM((BS * L,), jnp.int32),
            pltpu.VMEM((JC * L,), jnp.int32),
            pltpu.VMEM((JC * L,), jnp.int32),
            pltpu.VMEM((WQ, K), jnp.float32),
            pltpu.VMEM((WQ, K), jnp.float32),
            pltpu.VMEM((JB, WQ, M), jnp.float32),
            pltpu.SemaphoreType.DMA,
            pltpu.SemaphoreType.DMA,
            pltpu.SemaphoreType.DMA,
        ],
    )
    out = run(bank, label_rep, sel_flat)
    return out.reshape(bs, C, H, W, M).transpose(0, 4, 1, 2, 3)


# speedup vs baseline: 1.0084x; 1.0016x over previous
import jax
import jax.numpy as jnp
from jax import lax
from jax.experimental import pallas as pl
from jax.experimental.pallas import tpu as pltpu
from jax.experimental.pallas import tpu_sc as plsc

NUM_CLASSES = 10
K = 2000
N = 16
NC, NS, L = 2, 16, 16
NW = NC * NS
BS = 64
JB = 32
M = (NUM_CLASSES - 1) * N
CH = 3 * 32
WDIM = 32
WQ = 8
WSPLIT = WDIM // WQ
UNITS = CH * WSPLIT * (BS // JB)
UNITS_PER_W = UNITS // NW
JC = BS * NUM_CLASSES


def _gather_body(bank_hbm, labelrep_hbm, sel_hbm, out_hbm,
                 labelrep_v, sel_v, midx_v, slabA_v, slabB_v,
                 stage_v, semA, semB, osem):
    w = lax.axis_index("s") * NC + lax.axis_index("c")
    pltpu.sync_copy(labelrep_hbm, labelrep_v)
    pltpu.sync_copy(sel_hbm, sel_v)
    lanes = lax.iota(jnp.int32, L)

    @pl.loop(0, JC)
    def _tab(c2):
        j = c2 // NUM_CLASSES
        cls = c2 % NUM_CLASSES
        lbl = labelrep_v[pl.ds(j * L, L)]
        clsv = jnp.full((L,), cls, jnp.int32)
        gtv = (clsv > lbl).astype(jnp.int32)
        eqv = (clsv == lbl).astype(jnp.int32)
        midx_v[pl.ds(c2 * L, L)] = (clsv - gtv) * N + lanes + eqv * 4 * M

    slabs = (slabA_v, slabB_v)
    sems = (semA, semB)

    def slab_start(cls, ch, w0, b):
        pltpu.async_copy(bank_hbm.at[cls * CH + ch, pl.ds(w0, WQ), :],
                         slabs[b], sems[b])

    def slab_wait(ch, w0, b):
        pltpu.make_async_copy(bank_hbm.at[ch, pl.ds(w0, WQ), :],
                              slabs[b], sems[b]).wait()

    wfull = [jnp.full((L,), wr, jnp.int32) for wr in range(WQ)]

    @pl.loop(0, UNITS_PER_W)
    def _pos(i):
        u = w * UNITS_PER_W + i
        ch = u // (WSPLIT * 2)
        rest = u % (WSPLIT * 2)
        j0 = (rest % 2) * JB
        w0 = (rest // 2) * WQ

        slab_start(0, ch, w0, 0)

        @pl.when(i > 0)
        def _drain_prev():
            @pl.loop(0, JB)
            def _d(jj):
                pltpu.make_async_copy(
                    stage_v.at[0], out_hbm.at[0, 0, pl.ds(0, WQ), :],
                    osem).wait()

        for cls in range(NUM_CLASSES):
            b = cls % 2
            slab_wait(ch, w0, b)
            if cls + 1 < NUM_CLASSES:
                slab_start(cls + 1, ch, w0, 1 - b)

            last = cls == NUM_CLASSES - 1

            @plsc.parallel_loop(0, JB, unroll=2)
            def _sample(jj):
                j = j0 + jj
                base = (j * NUM_CLASSES + cls) * L
                kvec = sel_v[pl.ds(base, L)]
                midx = midx_v[pl.ds(base, L)]
                mk = midx < M
                jful = jnp.full((L,), jj, jnp.int32)
                for wr in range(WQ):
                    vals = plsc.load_gather(slabs[b], [wfull[wr], kvec])
                    plsc.store_scatter(stage_v, [jful, wfull[wr], midx],
                                       vals, mask=mk)
                if last:
                    pltpu.async_copy(
                        stage_v.at[jj],
                        out_hbm.at[j0 + jj, ch, pl.ds(w0, WQ), :], osem)

    _final_drain(stage_v, out_hbm, osem)


def _final_drain(stage_v, out_hbm, osem):
    @pl.loop(0, JB)
    def _d(jj):
        pltpu.make_async_copy(stage_v.at[0],
                              out_hbm.at[0, 0, pl.ds(0, WQ), :], osem).wait()


def kernel(x_s, label, confounder_queue, sel_idx):
    bs = x_s.shape[0]
    C, H, W = (confounder_queue.shape[2], confounder_queue.shape[3],
               confounder_queue.shape[4])
    bank = confounder_queue.transpose(0, 2, 3, 4, 1).reshape(
        NUM_CLASSES * C * H, W, K)
    sel_flat = sel_idx.reshape(-1).astype(jnp.int32)
    label_rep = jnp.repeat(label.astype(jnp.int32), L)
    mesh = plsc.VectorSubcoreMesh(
        core_axis_name="c", subcore_axis_name="s",
        num_cores=NC, num_subcores=NS)
    run = pl.kernel(
        _gather_body,
        out_type=jax.ShapeDtypeStruct((bs, C * H, W, M), jnp.float32),
        mesh=mesh,
        compiler_params=pltpu.CompilerParams(
            needs_layout_passes=False, use_tc_tiling_on_sc=True,
            disable_bounds_checks=True),
        scratch_types=[
            pltpu.VMEM((BS * L,), jnp.int32),
            pltpu.VMEM((JC * L,), jnp.int32),
            pltpu.VMEM((JC * L,), jnp.int32),
            pltpu.VMEM((WQ, K), jnp.float32),
            pltpu.VMEM((WQ, K), jnp.float32),
            pltpu.VMEM((JB, WQ, M), jnp.float32),
            pltpu.SemaphoreType.DMA,
            pltpu.SemaphoreType.DMA,
            pltpu.SemaphoreType.DMA,
        ],
    )
    out = run(bank, label_rep, sel_flat)
    return out.reshape(bs, C, H, W, M).transpose(0, 4, 1, 2, 3)
